# baseline (speedup 1.0000x reference)
import functools

import jax
import jax.numpy as jnp
from jax import lax
from jax.experimental import pallas as pl
from jax.experimental.pallas import tpu as pltpu
from jax.experimental.pallas import tpu_sc as plsc

LENGTH = 2048
N_PAIRS = 16
FLAT = 2 * N_PAIRS

ROWS = 32768
COLS = 1024
N_WORKERS = 32
ROWS_PER_W = ROWS // N_WORKERS

_mesh = plsc.VectorSubcoreMesh(core_axis_name="c", subcore_axis_name="s")


@functools.partial(
    pl.kernel,
    mesh=_mesh,
    out_type=(
        jax.ShapeDtypeStruct((ROWS, COLS), jnp.float32),
        jax.ShapeDtypeStruct((FLAT,), jnp.int32),
    ),
    scratch_types=[pltpu.VMEM((FLAT,), jnp.int32)],
)
def _trunc_sc(values_hbm, idx_hbm, out_values_hbm, out_idx_hbm, scratch):
    cid = lax.axis_index("c")
    sid = lax.axis_index("s")
    wid = sid * 2 + cid
    base = wid * ROWS_PER_W
    pltpu.sync_copy(
        values_hbm.at[pl.ds(base, ROWS_PER_W)],
        out_values_hbm.at[pl.ds(base, ROWS_PER_W)],
    )

    @pl.when(jnp.logical_and(cid == 0, sid == 0))
    def _():
        pltpu.sync_copy(idx_hbm, scratch)
        lane = lax.iota(jnp.int32, 16)
        even = lane - (lane & 1)
        for i in range(FLAT // 16):
            x = scratch[pl.ds(16 * i, 16)]
            b = x.at[even].get(mode="promise_in_bounds")
            scratch[pl.ds(16 * i, 16)] = jnp.minimum(x, b + LENGTH)
        pltpu.sync_copy(scratch, out_idx_hbm)


def kernel(values, indices):
    vals_out, idx_out = _trunc_sc(values, indices.reshape(FLAT))
    return (vals_out, idx_out.reshape(N_PAIRS, 2))


# speedup vs baseline: 39.8663x; 39.8663x over previous
import functools

import jax
import jax.numpy as jnp
from jax import lax
from jax.experimental import pallas as pl
from jax.experimental.pallas import tpu as pltpu
from jax.experimental.pallas import tpu_sc as plsc

LENGTH = 2048
N_PAIRS = 16
FLAT = 2 * N_PAIRS

_mesh = plsc.VectorSubcoreMesh(core_axis_name="c", subcore_axis_name="s")


@functools.partial(
    pl.kernel,
    mesh=_mesh,
    out_type=jax.ShapeDtypeStruct((FLAT,), jnp.int32),
    scratch_types=[pltpu.VMEM((FLAT,), jnp.int32)],
)
def _trunc_sc(idx_hbm, out_hbm, scratch):
    cid = lax.axis_index("c")
    sid = lax.axis_index("s")

    @pl.when(jnp.logical_and(cid == 0, sid == 0))
    def _():
        pltpu.sync_copy(idx_hbm, scratch)
        lane = lax.iota(jnp.int32, 16)
        even = lane - (lane & 1)
        for i in range(FLAT // 16):
            x = scratch[pl.ds(16 * i, 16)]
            b = x.at[even].get(mode="promise_in_bounds")
            scratch[pl.ds(16 * i, 16)] = jnp.minimum(x, b + LENGTH)
        pltpu.sync_copy(scratch, out_hbm)


def _copy_body(x_ref, o_ref):
    o_ref[...] = x_ref[...]


def _tc_copy(values):
    rows, cols = values.shape
    block = 1024
    return pl.pallas_call(
        _copy_body,
        grid=(rows // block,),
        in_specs=[pl.BlockSpec((block, cols), lambda i: (i, 0))],
        out_specs=pl.BlockSpec((block, cols), lambda i: (i, 0)),
        out_shape=jax.ShapeDtypeStruct(values.shape, values.dtype),
    )(values)


def kernel(values, indices):
    vals_out = _tc_copy(values)
    out = _trunc_sc(indices.reshape(FLAT))
    return (vals_out, out.reshape(N_PAIRS, 2))


# speedup vs baseline: 40.4024x; 1.0134x over previous
import functools

import jax
import jax.numpy as jnp
from jax import lax
from jax.experimental import pallas as pl
from jax.experimental.pallas import tpu as pltpu
from jax.experimental.pallas import tpu_sc as plsc

LENGTH = 2048
N_PAIRS = 16
FLAT = 2 * N_PAIRS

_mesh = plsc.VectorSubcoreMesh(
    core_axis_name="c", subcore_axis_name="s", num_cores=1, num_subcores=1
)


@functools.partial(
    pl.kernel,
    mesh=_mesh,
    out_type=jax.ShapeDtypeStruct((FLAT,), jnp.int32),
    scratch_types=[pltpu.VMEM((FLAT,), jnp.int32)],
)
def _trunc_sc(idx_hbm, out_hbm, scratch):
    cid = lax.axis_index("c")
    sid = lax.axis_index("s")

    @pl.when(jnp.logical_and(cid == 0, sid == 0))
    def _():
        pltpu.sync_copy(idx_hbm, scratch)
        lane = lax.iota(jnp.int32, 16)
        even = lane - (lane & 1)
        for i in range(FLAT // 16):
            x = scratch[pl.ds(16 * i, 16)]
            b = x.at[even].get(mode="promise_in_bounds")
            scratch[pl.ds(16 * i, 16)] = jnp.minimum(x, b + LENGTH)
        pltpu.sync_copy(scratch, out_hbm)


def _copy_body(x_ref, o_ref):
    o_ref[...] = x_ref[...]


def _tc_copy(values):
    rows, cols = values.shape
    block = 1024
    return pl.pallas_call(
        _copy_body,
        grid=(rows // block,),
        in_specs=[pl.BlockSpec((block, cols), lambda i: (i, 0))],
        out_specs=pl.BlockSpec((block, cols), lambda i: (i, 0)),
        out_shape=jax.ShapeDtypeStruct(values.shape, values.dtype),
    )(values)


def kernel(values, indices):
    vals_out = _tc_copy(values)
    out = _trunc_sc(indices.reshape(FLAT))
    return (vals_out, out.reshape(N_PAIRS, 2))
